# Initial kernel scaffold; baseline (speedup 1.0000x reference)
import jax
import jax.numpy as jnp
from jax.experimental import pallas as pl


def kernel(confidences, predictions, labels):
    raise NotImplementedError("write your pallas kernel here")


# speedup vs baseline: 1.0293x; 1.0293x over previous
import functools

import jax
import jax.numpy as jnp
from jax import lax
from jax.experimental import pallas as pl
from jax.experimental.pallas import tpu as pltpu
from jax.experimental.pallas import tpu_sc as plsc

N_BINS = 15
_NW = 32
_LANES = 16


def _sc_hist(conf, pred, lab, pw):
    nchunk = pw // _LANES
    mesh = plsc.VectorSubcoreMesh(core_axis_name="c", subcore_axis_name="s")

    @functools.partial(
        pl.kernel, mesh=mesh,
        out_type=jax.ShapeDtypeStruct((_NW, 3 * N_BINS, _LANES), jnp.float32),
        scratch_types=[
            pltpu.VMEM((pw,), jnp.float32),
            pltpu.VMEM((pw,), jnp.int32),
            pltpu.VMEM((pw,), jnp.int32),
            pltpu.VMEM((3 * N_BINS, _LANES), jnp.float32),
        ],
        compiler_params=pltpu.CompilerParams(needs_layout_passes=False),
    )
    def k(conf_hbm, pred_hbm, lab_hbm, out_hbm,
          conf_v, pred_v, lab_v, hist_v):
        wid = lax.axis_index("s") * 2 + lax.axis_index("c")
        base = wid * pw
        pltpu.sync_copy(conf_hbm.at[pl.ds(base, pw)], conf_v)
        pltpu.sync_copy(pred_hbm.at[pl.ds(base, pw)], pred_v)
        pltpu.sync_copy(lab_hbm.at[pl.ds(base, pw)], lab_v)

        zf = jnp.zeros((_LANES,), jnp.float32)
        for r in range(3 * N_BINS):
            hist_v[r] = zf
        col = lax.iota(jnp.int32, _LANES)
        onesf = jnp.ones((_LANES,), jnp.float32)
        i0 = jnp.zeros((_LANES,), jnp.int32)
        i1 = jnp.ones((_LANES,), jnp.int32)
        i14 = jnp.full((_LANES,), N_BINS - 1, jnp.int32)
        i15 = jnp.full((_LANES,), N_BINS, jnp.int32)
        i30 = jnp.full((_LANES,), 2 * N_BINS, jnp.int32)
        f15 = jnp.full((_LANES,), 15.0, jnp.float32)
        stepv = jnp.full((_LANES,), 1.0, jnp.float32) / f15

        def body(i, carry):
            s = i * _LANES
            c = conf_v[pl.ds(s, _LANES)]
            p = pred_v[pl.ds(s, _LANES)]
            l = lab_v[pl.ds(s, _LANES)]
            acc = jnp.where(p == l, onesf, zf)
            t = (c * f15).astype(jnp.int32)
            idx0 = jnp.minimum(jnp.maximum(t, i0), i14)
            blo = idx0.astype(jnp.float32) * stepv
            bhi = (idx0 + i1).astype(jnp.float32) * stepv
            step1 = jnp.where(c > blo, i1, i0)
            step2 = jnp.where(c > bhi, i1, i0)
            idx = idx0 - i1 + step1 + step2
            valid = (idx >= i0) & (idx <= i14)
            row = jnp.minimum(jnp.maximum(idx, i0), i14)
            plsc.addupdate_scatter(hist_v, [row, col], onesf, mask=valid)
            plsc.addupdate_scatter(hist_v, [row + i15, col], c, mask=valid)
            plsc.addupdate_scatter(hist_v, [row + i30, col], acc, mask=valid)
            return carry

        lax.fori_loop(0, nchunk, body, 0)
        pltpu.sync_copy(hist_v, out_hbm.at[wid])

    return k(conf, pred, lab)


def _ece_body(p_ref, o_ref, *, n_total):
    x = p_ref[...]
    rows = jnp.sum(x, axis=1, keepdims=True)
    cnt = rows[0:N_BINS]
    sconf = rows[N_BINS:2 * N_BINS]
    sacc = rows[2 * N_BINS:3 * N_BINS]
    safe = jnp.maximum(cnt, 1.0)
    prop = cnt / n_total
    contrib = jnp.abs(sconf / safe - sacc / safe) * prop
    ece = jnp.sum(jnp.where(prop > 0.0, contrib, 0.0))
    o_ref[...] = jnp.full((8, 128), ece, jnp.float32)


def kernel(confidences, predictions, labels):
    n = confidences.shape[0]
    pw = -(-n // (_NW * _LANES)) * _LANES
    npad = pw * _NW
    conf = jnp.pad(confidences.astype(jnp.float32), (0, npad - n))
    pred = jnp.pad(predictions.astype(jnp.int32), (0, npad - n))
    lab = jnp.pad(labels.astype(jnp.int32), (0, npad - n))

    partials = _sc_hist(conf, pred, lab, pw)
    flat = partials.transpose(1, 0, 2).reshape(3 * N_BINS, _NW * _LANES)

    ece = pl.pallas_call(
        functools.partial(_ece_body, n_total=float(n)),
        out_shape=jax.ShapeDtypeStruct((8, 128), jnp.float32),
    )(flat)
    return ece[0, 0:1]


# speedup vs baseline: 1.9461x; 1.8908x over previous
import functools

import jax
import jax.numpy as jnp
from jax import lax
from jax.experimental import pallas as pl
from jax.experimental.pallas import tpu as pltpu
from jax.experimental.pallas import tpu_sc as plsc

N_BINS = 15
_NW = 32
_LANES = 16


def _sc_hist(conf, pred, lab, ch, tail):
    nchunk = ch // _LANES
    ntail = tail // _LANES
    pw = ch + tail
    mesh = plsc.VectorSubcoreMesh(core_axis_name="c", subcore_axis_name="s")

    hsz = 3 * N_BINS * _LANES

    @functools.partial(
        pl.kernel, mesh=mesh,
        out_type=jax.ShapeDtypeStruct((_NW * hsz,), jnp.float32),
        scratch_types=[
            pltpu.VMEM((pw,), jnp.float32),
            pltpu.VMEM((pw,), jnp.int32),
            pltpu.VMEM((pw,), jnp.int32),
            pltpu.VMEM((hsz,), jnp.float32),
        ],
        compiler_params=pltpu.CompilerParams(needs_layout_passes=False),
    )
    def k(conf_hbm, pred_hbm, lab_hbm, out_hbm,
          conf_v, pred_v, lab_v, hist_v):
        wid = lax.axis_index("s") * 2 + lax.axis_index("c")
        base = wid * ch
        pltpu.sync_copy(conf_hbm.at[pl.ds(base, ch)], conf_v.at[pl.ds(0, ch)])
        pltpu.sync_copy(pred_hbm.at[pl.ds(base, ch)], pred_v.at[pl.ds(0, ch)])
        pltpu.sync_copy(lab_hbm.at[pl.ds(base, ch)], lab_v.at[pl.ds(0, ch)])
        is_last = wid == _NW - 1
        if tail:
            @pl.when(is_last)
            def _():
                tbase = _NW * ch
                pltpu.sync_copy(conf_hbm.at[pl.ds(tbase, tail)],
                                conf_v.at[pl.ds(ch, tail)])
                pltpu.sync_copy(pred_hbm.at[pl.ds(tbase, tail)],
                                pred_v.at[pl.ds(ch, tail)])
                pltpu.sync_copy(lab_hbm.at[pl.ds(tbase, tail)],
                                lab_v.at[pl.ds(ch, tail)])

        zf = jnp.zeros((_LANES,), jnp.float32)
        for r in range(3 * N_BINS):
            hist_v[pl.ds(r * _LANES, _LANES)] = zf
        col = lax.iota(jnp.int32, _LANES)
        onesf = jnp.ones((_LANES,), jnp.float32)
        i0 = jnp.zeros((_LANES,), jnp.int32)
        i1 = jnp.ones((_LANES,), jnp.int32)
        i14 = jnp.full((_LANES,), N_BINS - 1, jnp.int32)
        i240 = jnp.full((_LANES,), N_BINS * _LANES, jnp.int32)
        i480 = jnp.full((_LANES,), 2 * N_BINS * _LANES, jnp.int32)
        f15 = jnp.full((_LANES,), 15.0, jnp.float32)
        stepv = jnp.full((_LANES,), 1.0, jnp.float32) / f15

        def chunk(s):
            c = conf_v[pl.ds(s, _LANES)]
            p = pred_v[pl.ds(s, _LANES)]
            l = lab_v[pl.ds(s, _LANES)]
            acc = jnp.where(p == l, onesf, zf)
            t = (c * f15).astype(jnp.int32)
            idx0 = jnp.minimum(jnp.maximum(t, i0), i14)
            blo = idx0.astype(jnp.float32) * stepv
            bhi = (idx0 + i1).astype(jnp.float32) * stepv
            step1 = jnp.where(c > blo, i1, i0)
            step2 = jnp.where(c > bhi, i1, i0)
            idx = idx0 - i1 + step1 + step2
            valid = idx >= i0
            row = jnp.minimum(jnp.maximum(idx, i0), i14)
            flat = row * _LANES + col
            plsc.addupdate_scatter(hist_v, [flat], onesf, mask=valid)
            plsc.addupdate_scatter(hist_v, [flat + i240], c, mask=valid)
            plsc.addupdate_scatter(hist_v, [flat + i480], acc, mask=valid)

        @plsc.parallel_loop(0, nchunk * _LANES, step=_LANES, unroll=8)
        def _(s):
            chunk(s)

        if ntail:
            @pl.when(is_last)
            def _():
                @plsc.parallel_loop(ch, ch + ntail * _LANES, step=_LANES)
                def _(s):
                    chunk(s)

        pltpu.sync_copy(hist_v, out_hbm.at[pl.ds(wid * hsz, hsz)])

    return k(conf, pred, lab)


def _ece_body(p_ref, o_ref, *, n_total):
    x = p_ref[...]
    s = jnp.sum(x, axis=0)
    rows = jnp.sum(s, axis=1, keepdims=True)
    cnt = rows[0:N_BINS]
    sconf = rows[N_BINS:2 * N_BINS]
    sacc = rows[2 * N_BINS:3 * N_BINS]
    safe = jnp.maximum(cnt, 1.0)
    prop = cnt / n_total
    contrib = jnp.abs(sconf / safe - sacc / safe) * prop
    ece = jnp.sum(jnp.where(prop > 0.0, contrib, 0.0))
    o_ref[...] = jnp.full((8, 128), ece, jnp.float32)


def kernel(confidences, predictions, labels):
    n = confidences.shape[0]
    conf = confidences.astype(jnp.float32)
    pred = predictions.astype(jnp.int32)
    lab = labels.astype(jnp.int32)
    if n % _LANES:
        npad = -(-n // _LANES) * _LANES
        conf = jnp.pad(conf, (0, npad - n))
        pred = jnp.pad(pred, (0, npad - n))
        lab = jnp.pad(lab, (0, npad - n))
        m = npad
    else:
        m = n
    ch = (m // (_NW * _LANES)) * _LANES
    tail = m - _NW * ch

    flat = _sc_hist(conf, pred, lab, ch, tail)
    parts = flat.reshape(_NW, 3 * N_BINS, _LANES)

    ece = pl.pallas_call(
        functools.partial(_ece_body, n_total=float(n)),
        out_shape=jax.ShapeDtypeStruct((8, 128), jnp.float32),
    )(parts)
    return ece[0, 0:1]
